# baseline (speedup 1.0000x reference)
import functools

import jax
import jax.numpy as jnp
from jax import lax
from jax.experimental import pallas as pl
from jax.experimental.pallas import tpu as pltpu
from jax.experimental.pallas import tpu_sc as plsc

_NC = 2
_NS = 16
_L = 16
_CH = 128


_DW = 128
_G = 8


def _make_sc_degree(n, n_acc, e_rows):
    ecw = e_rows // (_NC * _NS)
    rpt = n_acc // _NS

    def body(dst2_hbm, ones_hbm, zeros_hbm, out_hbm, ones_v, idx_v, acc_sh,
             ssem):
        c = lax.axis_index("c")
        s = lax.axis_index("s")
        wid = s * _NC + c

        pltpu.sync_copy(ones_hbm, ones_v)
        pltpu.sync_copy(zeros_hbm, acc_sh.at[pl.ds(s * rpt, rpt)])
        plsc.subcore_barrier()

        base = wid * ecw

        @pl.loop(0, ecw // _G)
        def _grp(g):
            r0 = base + g * _G
            pltpu.sync_copy(dst2_hbm.at[pl.ds(r0, _G)], idx_v)
            for j in range(_G):
                pltpu.make_async_copy(
                    ones_v, acc_sh.at[idx_v.at[j]], ssem).start(add=True)
            for j in range(_G):
                pltpu.make_async_copy(
                    ones_v, acc_sh.at[idx_v.at[j]], ssem).wait()

        plsc.subcore_barrier()
        cpt = (n // _NS) // 8 * 8
        tail = n - _NS * cpt
        pltpu.sync_copy(acc_sh.at[pl.ds(s * cpt, cpt)],
                        out_hbm.at[pl.ds(c * n + s * cpt, cpt)])
        if tail:
            @pl.when(s == _NS - 1)
            def _tail():
                pltpu.sync_copy(acc_sh.at[pl.ds(n - tail, tail)],
                                out_hbm.at[pl.ds(c * n + n - tail, tail)])

    mesh = plsc.VectorSubcoreMesh(core_axis_name="c", subcore_axis_name="s")
    return pl.kernel(
        body,
        out_type=jax.ShapeDtypeStruct((_NC * n, _DW), jnp.float32),
        mesh=mesh,
        scratch_types=[
            pltpu.VMEM((_CH, _DW), jnp.float32),
            pltpu.VMEM((_G, _CH), jnp.int32),
            pltpu.VMEM_SHARED((n_acc, _DW), jnp.float32),
            pltpu.SemaphoreType.DMA,
        ],
    )


def _make_sc_agg(n, n_acc, e_rows, d):
    crows = e_rows // _NS
    rpt = n_acc // _NS

    def body(table_hbm, srcs_hbm, dst_hbm, zeros_hbm, out_hbm,
             sidx_v, didx_v, rows_v, acc_sh, gsem0, gsem1, ssem0, ssem1):
        c = lax.axis_index("c")
        s = lax.axis_index("s")
        gsem = (gsem0, gsem1)
        ssem = (ssem0, ssem1)

        pltpu.sync_copy(zeros_hbm, acc_sh.at[pl.ds(s * rpt, rpt)])
        plsc.subcore_barrier()

        base = s * crows

        def gather(j, p):
            return pltpu.make_async_copy(
                table_hbm.at[sidx_v.at[j]], rows_v.at[p], gsem[p])

        def scatter(j, p):
            return pltpu.make_async_copy(
                rows_v.at[p], acc_sh.at[didx_v.at[j]], ssem[p])

        @pl.loop(0, crows // _G)
        def _grp(g):
            r0 = base + g * _G
            pltpu.sync_copy(srcs_hbm.at[pl.ds(c * e_rows + r0, _G)], sidx_v)
            pltpu.sync_copy(dst_hbm.at[pl.ds(r0, _G)], didx_v)
            gather(0, 0).start()
            for j in range(_G):
                p = j % 2
                gather(j, p).wait()
                scatter(j, p).start(add=True)
                if j + 1 < _G:
                    if j >= 1:
                        scatter(j - 1, 1 - p).wait()
                    gather(j + 1, 1 - p).start()
            scatter(_G - 2, 0).wait()
            scatter(_G - 1, 1).wait()

        plsc.subcore_barrier()
        cpt = (n // _NS) // 8 * 8
        tail = n - _NS * cpt
        pltpu.sync_copy(acc_sh.at[pl.ds(s * cpt, cpt)],
                        out_hbm.at[pl.ds(c * n + s * cpt, cpt)])
        if tail:
            @pl.when(s == _NS - 1)
            def _tail():
                pltpu.sync_copy(acc_sh.at[pl.ds(n - tail, tail)],
                                out_hbm.at[pl.ds(c * n + n - tail, tail)])

    mesh = plsc.VectorSubcoreMesh(core_axis_name="c", subcore_axis_name="s")
    return pl.kernel(
        body,
        out_type=jax.ShapeDtypeStruct((_NC * n, d), jnp.float32),
        mesh=mesh,
        scratch_types=[
            pltpu.VMEM((_G, _CH), jnp.int32),
            pltpu.VMEM((_G, _CH), jnp.int32),
            pltpu.VMEM((2, _CH, d), jnp.float32),
            pltpu.VMEM_SHARED((n_acc, d), jnp.float32),
            pltpu.SemaphoreType.DMA,
            pltpu.SemaphoreType.DMA,
            pltpu.SemaphoreType.DMA,
            pltpu.SemaphoreType.DMA,
        ],
    )


def _tc1_body(x_r, topo_r, d0_r, d1_r, wt1_r, bt1_r, wt2_r, bt2_r, w1_r,
              out_r, dinv_r):
    tz = jnp.maximum(
        jnp.dot(topo_r[...], wt1_r[...], preferred_element_type=jnp.float32)
        + bt1_r[...], 0.0)
    tz = jnp.maximum(
        jnp.dot(tz, wt2_r[...], preferred_element_type=jnp.float32)
        + bt2_r[...], 0.0)
    h = jnp.concatenate([x_r[...], tz], axis=1)
    hw = jnp.dot(h, w1_r[...], preferred_element_type=jnp.float32)
    deg = d0_r[:, 0:1] + d1_r[:, 0:1] + 1.0
    dinv = lax.rsqrt(deg)
    hwp = hw * dinv
    half = hw.shape[1] // 2
    out_r[0] = hwp[:, :half]
    out_r[1] = hwp[:, half:]
    dinv_r[...] = dinv


def _tc_mid_body(agg_r, hwp_r, dinv_r, b_r, w_r, out_r):
    sfull = jnp.concatenate(
        [agg_r[0] + hwp_r[0], agg_r[1] + hwp_r[1]], axis=1)
    hcur = jnp.maximum(dinv_r[...] * sfull + b_r[...], 0.0)
    hw = jnp.dot(hcur, w_r[...], preferred_element_type=jnp.float32)
    hwp = hw * dinv_r[...]
    half = hw.shape[1] // 2
    out_r[0] = hwp[:, :half]
    out_r[1] = hwp[:, half:]


def _tc_out_body(agg_r, hwp_r, dinv_r, b_r, wout_r, bout_r, out_r):
    sfull = jnp.concatenate(
        [agg_r[0] + hwp_r[0], agg_r[1] + hwp_r[1]], axis=1)
    hcur = jnp.maximum(dinv_r[...] * sfull + b_r[...], 0.0)
    o = jnp.dot(hcur, wout_r[...], preferred_element_type=jnp.float32)
    o = o + bout_r[...]
    out_r[...] = 1.0 / (1.0 + jnp.exp(-o))


def _full2(a):
    return pl.BlockSpec(a.shape, lambda i: (0, 0))


def kernel(x, edge_index, topo, Wt1, bt1, Wt2, bt2, W1, b1, W2, b2,
           Wout, bout):
    n, d_in = x.shape
    hid = W1.shape[1]
    half = hid // 2
    src = edge_index[0]
    dst = edge_index[1]
    e = src.shape[0]

    quant = _NC * _NS * _CH * _G
    e_pad = -(-e // quant) * quant
    pad = e_pad - e
    e_rows = e_pad // _CH
    n_acc = -(-(n + 1) // (_NS * 8)) * (_NS * 8)

    src_p = jnp.concatenate([src, jnp.zeros((pad,), src.dtype)])
    dst_p = jnp.concatenate([dst, jnp.full((pad,), n, dst.dtype)])
    dst2 = dst_p.reshape(e_rows, _CH)
    srcs2 = jnp.concatenate([src_p, src_p + n]).reshape(2 * e_rows, _CH)

    rpt_deg = n_acc // _NS
    deg_out = _make_sc_degree(n, n_acc, e_rows)(
        dst2, jnp.ones((_CH, _DW), jnp.float32),
        jnp.zeros((rpt_deg, _DW), jnp.float32))
    d0 = deg_out[0:n, 0:_L]
    d1 = deg_out[n:2 * n, 0:_L]

    B = 1000
    grid = (n // B,)
    row = lambda shp: pl.BlockSpec(shp, lambda i: (i, 0))
    row3 = lambda shp: pl.BlockSpec(shp, lambda i: (0, i, 0))
    table1, dinv = pl.pallas_call(
        _tc1_body,
        grid=grid,
        in_specs=[
            row((B, d_in)), row((B, topo.shape[1])),
            row((B, _L)), row((B, _L)),
            _full2(Wt1), _full2(bt1.reshape(1, -1)),
            _full2(Wt2), _full2(bt2.reshape(1, -1)),
            _full2(W1),
        ],
        out_specs=[row3((2, B, half)), row((B, 1))],
        out_shape=[
            jax.ShapeDtypeStruct((2, n, half), jnp.float32),
            jax.ShapeDtypeStruct((n, 1), jnp.float32),
        ],
    )(x, topo, d0, d1, Wt1, bt1.reshape(1, -1), Wt2, bt2.reshape(1, -1), W1)

    agg_call = _make_sc_agg(n, n_acc, e_rows, half)
    zeros_agg = jnp.zeros((n_acc // _NS, half), jnp.float32)

    agg1 = agg_call(table1.reshape(2 * n, half), srcs2, dst2, zeros_agg)
    agg1 = agg1.reshape(2, n, half)
    table2 = pl.pallas_call(
        _tc_mid_body,
        grid=grid,
        in_specs=[
            row3((2, B, half)), row3((2, B, half)), row((B, 1)),
            _full2(b1.reshape(1, -1)), _full2(W2),
        ],
        out_specs=row3((2, B, half)),
        out_shape=jax.ShapeDtypeStruct((2, n, half), jnp.float32),
    )(agg1, table1, dinv, b1.reshape(1, -1), W2)

    agg2 = agg_call(table2.reshape(2 * n, half), srcs2, dst2, zeros_agg)
    agg2 = agg2.reshape(2, n, half)
    out = pl.pallas_call(
        _tc_out_body,
        grid=grid,
        in_specs=[
            row3((2, B, half)), row3((2, B, half)), row((B, 1)),
            _full2(b2.reshape(1, -1)), _full2(Wout),
            _full2(bout.reshape(1, -1)),
        ],
        out_specs=row((B, 1)),
        out_shape=jax.ShapeDtypeStruct((n, 1), jnp.float32),
    )(agg2, table2, dinv, b2.reshape(1, -1), Wout, bout.reshape(1, -1))
    return out


# speedup vs baseline: 1.0200x; 1.0200x over previous
import functools

import jax
import jax.numpy as jnp
from jax import lax
from jax.experimental import pallas as pl
from jax.experimental.pallas import tpu as pltpu
from jax.experimental.pallas import tpu_sc as plsc

_NC = 2
_NS = 16
_L = 16
_CH = 128


_DW = 128
_G = 16


def _make_sc_degree(n, n_acc, e_rows):
    ecw = e_rows // (_NC * _NS)
    rpt = n_acc // _NS

    def body(dst2_hbm, ones_hbm, zeros_hbm, out_hbm, ones_v, idx_v, acc_sh,
             ssem):
        c = lax.axis_index("c")
        s = lax.axis_index("s")
        wid = s * _NC + c

        pltpu.sync_copy(ones_hbm, ones_v)
        pltpu.sync_copy(zeros_hbm, acc_sh.at[pl.ds(s * rpt, rpt)])
        plsc.subcore_barrier()

        base = wid * ecw

        @pl.loop(0, ecw // _G)
        def _grp(g):
            r0 = base + g * _G
            pltpu.sync_copy(dst2_hbm.at[pl.ds(r0, _G)], idx_v)
            for j in range(_G):
                pltpu.make_async_copy(
                    ones_v, acc_sh.at[idx_v.at[j]], ssem).start(add=True)
            for j in range(_G):
                pltpu.make_async_copy(
                    ones_v, acc_sh.at[idx_v.at[j]], ssem).wait()

        plsc.subcore_barrier()
        cpt = (n // _NS) // 8 * 8
        tail = n - _NS * cpt
        pltpu.sync_copy(acc_sh.at[pl.ds(s * cpt, cpt)],
                        out_hbm.at[pl.ds(c * n + s * cpt, cpt)])
        if tail:
            @pl.when(s == _NS - 1)
            def _tail():
                pltpu.sync_copy(acc_sh.at[pl.ds(n - tail, tail)],
                                out_hbm.at[pl.ds(c * n + n - tail, tail)])

    mesh = plsc.VectorSubcoreMesh(core_axis_name="c", subcore_axis_name="s")
    return pl.kernel(
        body,
        out_type=jax.ShapeDtypeStruct((_NC * n, _DW), jnp.float32),
        mesh=mesh,
        scratch_types=[
            pltpu.VMEM((_CH, _DW), jnp.float32),
            pltpu.VMEM((_G, _CH), jnp.int32),
            pltpu.VMEM_SHARED((n_acc, _DW), jnp.float32),
            pltpu.SemaphoreType.DMA,
        ],
    )


def _make_sc_agg(n, n_acc, e_rows, d):
    crows = e_rows // _NS
    rpt = n_acc // _NS

    def body(table_hbm, srcs_hbm, dst_hbm, zeros_hbm, out_hbm,
             sidx_v, didx_v, rows_v, acc_sh, gsem0, gsem1, ssem0, ssem1):
        c = lax.axis_index("c")
        s = lax.axis_index("s")
        gsem = (gsem0, gsem1)
        ssem = (ssem0, ssem1)

        pltpu.sync_copy(zeros_hbm, acc_sh.at[pl.ds(s * rpt, rpt)])
        plsc.subcore_barrier()

        base = s * crows

        def gather(j, p):
            return pltpu.make_async_copy(
                table_hbm.at[sidx_v.at[j]], rows_v.at[p], gsem[p])

        def scatter(j, p):
            return pltpu.make_async_copy(
                rows_v.at[p], acc_sh.at[didx_v.at[j]], ssem[p])

        @pl.loop(0, crows // _G)
        def _grp(g):
            r0 = base + g * _G
            pltpu.sync_copy(srcs_hbm.at[pl.ds(c * e_rows + r0, _G)], sidx_v)
            pltpu.sync_copy(dst_hbm.at[pl.ds(r0, _G)], didx_v)
            gather(0, 0).start()
            for j in range(_G):
                p = j % 2
                gather(j, p).wait()
                scatter(j, p).start(add=True)
                if j + 1 < _G:
                    if j >= 1:
                        scatter(j - 1, 1 - p).wait()
                    gather(j + 1, 1 - p).start()
            scatter(_G - 2, 0).wait()
            scatter(_G - 1, 1).wait()

        plsc.subcore_barrier()
        cpt = (n // _NS) // 8 * 8
        tail = n - _NS * cpt
        pltpu.sync_copy(acc_sh.at[pl.ds(s * cpt, cpt)],
                        out_hbm.at[pl.ds(c * n + s * cpt, cpt)])
        if tail:
            @pl.when(s == _NS - 1)
            def _tail():
                pltpu.sync_copy(acc_sh.at[pl.ds(n - tail, tail)],
                                out_hbm.at[pl.ds(c * n + n - tail, tail)])

    mesh = plsc.VectorSubcoreMesh(core_axis_name="c", subcore_axis_name="s")
    return pl.kernel(
        body,
        out_type=jax.ShapeDtypeStruct((_NC * n, d), jnp.float32),
        mesh=mesh,
        scratch_types=[
            pltpu.VMEM((_G, _CH), jnp.int32),
            pltpu.VMEM((_G, _CH), jnp.int32),
            pltpu.VMEM((2, _CH, d), jnp.float32),
            pltpu.VMEM_SHARED((n_acc, d), jnp.float32),
            pltpu.SemaphoreType.DMA,
            pltpu.SemaphoreType.DMA,
            pltpu.SemaphoreType.DMA,
            pltpu.SemaphoreType.DMA,
        ],
    )


def _tc1_body(x_r, topo_r, d0_r, d1_r, wt1_r, bt1_r, wt2_r, bt2_r, w1_r,
              out_r, dinv_r):
    tz = jnp.maximum(
        jnp.dot(topo_r[...], wt1_r[...], preferred_element_type=jnp.float32)
        + bt1_r[...], 0.0)
    tz = jnp.maximum(
        jnp.dot(tz, wt2_r[...], preferred_element_type=jnp.float32)
        + bt2_r[...], 0.0)
    h = jnp.concatenate([x_r[...], tz], axis=1)
    hw = jnp.dot(h, w1_r[...], preferred_element_type=jnp.float32)
    deg = d0_r[:, 0:1] + d1_r[:, 0:1] + 1.0
    dinv = lax.rsqrt(deg)
    hwp = hw * dinv
    half = hw.shape[1] // 2
    out_r[0] = hwp[:, :half]
    out_r[1] = hwp[:, half:]
    dinv_r[...] = dinv


def _tc_mid_body(agg_r, hwp_r, dinv_r, b_r, w_r, out_r):
    sfull = jnp.concatenate(
        [agg_r[0] + hwp_r[0], agg_r[1] + hwp_r[1]], axis=1)
    hcur = jnp.maximum(dinv_r[...] * sfull + b_r[...], 0.0)
    hw = jnp.dot(hcur, w_r[...], preferred_element_type=jnp.float32)
    hwp = hw * dinv_r[...]
    half = hw.shape[1] // 2
    out_r[0] = hwp[:, :half]
    out_r[1] = hwp[:, half:]


def _tc_out_body(agg_r, hwp_r, dinv_r, b_r, wout_r, bout_r, out_r):
    sfull = jnp.concatenate(
        [agg_r[0] + hwp_r[0], agg_r[1] + hwp_r[1]], axis=1)
    hcur = jnp.maximum(dinv_r[...] * sfull + b_r[...], 0.0)
    o = jnp.dot(hcur, wout_r[...], preferred_element_type=jnp.float32)
    o = o + bout_r[...]
    out_r[...] = 1.0 / (1.0 + jnp.exp(-o))


def _full2(a):
    return pl.BlockSpec(a.shape, lambda i: (0, 0))


def kernel(x, edge_index, topo, Wt1, bt1, Wt2, bt2, W1, b1, W2, b2,
           Wout, bout):
    n, d_in = x.shape
    hid = W1.shape[1]
    half = hid // 2
    src = edge_index[0]
    dst = edge_index[1]
    e = src.shape[0]

    quant = _NC * _NS * _CH * _G
    e_pad = -(-e // quant) * quant
    pad = e_pad - e
    e_rows = e_pad // _CH
    n_acc = -(-(n + 1) // (_NS * 8)) * (_NS * 8)

    src_p = jnp.concatenate([src, jnp.zeros((pad,), src.dtype)])
    dst_p = jnp.concatenate([dst, jnp.full((pad,), n, dst.dtype)])
    dst2 = dst_p.reshape(e_rows, _CH)
    srcs2 = jnp.concatenate([src_p, src_p + n]).reshape(2 * e_rows, _CH)

    rpt_deg = n_acc // _NS
    deg_out = _make_sc_degree(n, n_acc, e_rows)(
        dst2, jnp.ones((_CH, _DW), jnp.float32),
        jnp.zeros((rpt_deg, _DW), jnp.float32))
    d0 = deg_out[0:n, 0:_L]
    d1 = deg_out[n:2 * n, 0:_L]

    B = 1000
    grid = (n // B,)
    row = lambda shp: pl.BlockSpec(shp, lambda i: (i, 0))
    row3 = lambda shp: pl.BlockSpec(shp, lambda i: (0, i, 0))
    table1, dinv = pl.pallas_call(
        _tc1_body,
        grid=grid,
        in_specs=[
            row((B, d_in)), row((B, topo.shape[1])),
            row((B, _L)), row((B, _L)),
            _full2(Wt1), _full2(bt1.reshape(1, -1)),
            _full2(Wt2), _full2(bt2.reshape(1, -1)),
            _full2(W1),
        ],
        out_specs=[row3((2, B, half)), row((B, 1))],
        out_shape=[
            jax.ShapeDtypeStruct((2, n, half), jnp.float32),
            jax.ShapeDtypeStruct((n, 1), jnp.float32),
        ],
    )(x, topo, d0, d1, Wt1, bt1.reshape(1, -1), Wt2, bt2.reshape(1, -1), W1)

    agg_call = _make_sc_agg(n, n_acc, e_rows, half)
    zeros_agg = jnp.zeros((n_acc // _NS, half), jnp.float32)

    agg1 = agg_call(table1.reshape(2 * n, half), srcs2, dst2, zeros_agg)
    agg1 = agg1.reshape(2, n, half)
    table2 = pl.pallas_call(
        _tc_mid_body,
        grid=grid,
        in_specs=[
            row3((2, B, half)), row3((2, B, half)), row((B, 1)),
            _full2(b1.reshape(1, -1)), _full2(W2),
        ],
        out_specs=row3((2, B, half)),
        out_shape=jax.ShapeDtypeStruct((2, n, half), jnp.float32),
    )(agg1, table1, dinv, b1.reshape(1, -1), W2)

    agg2 = agg_call(table2.reshape(2 * n, half), srcs2, dst2, zeros_agg)
    agg2 = agg2.reshape(2, n, half)
    out = pl.pallas_call(
        _tc_out_body,
        grid=grid,
        in_specs=[
            row3((2, B, half)), row3((2, B, half)), row((B, 1)),
            _full2(b2.reshape(1, -1)), _full2(Wout),
            _full2(bout.reshape(1, -1)),
        ],
        out_specs=row((B, 1)),
        out_shape=jax.ShapeDtypeStruct((n, 1), jnp.float32),
    )(agg2, table2, dinv, b2.reshape(1, -1), Wout, bout.reshape(1, -1))
    return out
